# Initial kernel scaffold; baseline (speedup 1.0000x reference)
import jax
import jax.numpy as jnp
from jax.experimental import pallas as pl


def kernel(h_lnc, h_mi, h_m, src0, dst0, src1, dst1, src2, dst2, src3, dst3, src4, dst4, src5, dst5, basis1, coeff1, loop1, bias1, basis2, coeff2, loop2, bias2, dec_W1, dec_b1, dec_W2, dec_b2):
    raise NotImplementedError("write your pallas kernel here")


# speedup vs baseline: 2.2205x; 2.2205x over previous
import functools

import jax
import jax.numpy as jnp
from jax import lax
from jax.experimental import pallas as pl
from jax.experimental.pallas import tpu as pltpu
from jax.experimental.pallas import tpu_sc as plsc

N_LNC, N_MI, N_M = 10000, 5000, 20000
N_TOT = N_LNC + N_MI + N_M
OFF = {"lnc": 0, "mi": N_LNC, "m": N_LNC + N_MI}
FEAT = 256
OUT = 128
E = 50000
NB = 4
NR = 6

NCORES = 2
NS = 16
W = 128
NPCOL = FEAT // W
CHUNK = 64
EPT = 3200
E_PAD = NS * EPT
NCHUNK = EPT // CHUNK
ONES_ROWS = 2048

GROUPS = (
    dict(name="lnc", n=N_LNC, nh=1, h_sz=(N_LNC,), nr_acc=10112, r_blk=1000,
         rels=(1, 5), srct=("mi", "m")),
    dict(name="mi", n=N_MI, nh=1, h_sz=(N_MI,), nr_acc=5120, r_blk=1000,
         rels=(0, 3), srct=("lnc", "m")),
    dict(name="m", n=N_M, nh=2, h_sz=(12800, 7200), nr_acc=12928, r_blk=400,
         rels=(2, 4), srct=("mi", "lnc")),
)
H_SPLIT_M = 12800


def _seg_body(nr_acc, nh, with_deg, *refs):
    rpt = nr_acc // NS
    if with_deg:
        (table, ones_tab, srcidx, dstidx, degidx, zeros_hbm,
         s_out, deg_out, acc, srci, dsti, rows, sem) = refs
    else:
        (table, srcidx, dstidx, zeros_hbm,
         s_out, acc, srci, dsti, rows, sem) = refs
    c = lax.axis_index("c")
    s = lax.axis_index("s")
    ebase = s * EPT
    rbase = s * rpt

    def run_pass(tab, src_at, dst_at, out_at):
        pltpu.sync_copy(zeros_hbm, acc.at[pl.ds(rbase, rpt)])
        plsc.subcore_barrier()

        def chunk_body(j, carry):
            base = ebase + j * CHUNK
            pltpu.sync_copy(src_at(base), srci)
            pltpu.sync_copy(dst_at(base), dsti)
            pltpu.async_copy(tab.at[srci], rows, sem).wait()
            pltpu.sync_copy(rows, acc.at[dsti], add=True)
            return carry

        lax.fori_loop(0, NCHUNK, chunk_body, 0)
        plsc.subcore_barrier()
        pltpu.sync_copy(acc.at[pl.ds(rbase, rpt)], out_at)

    for hh in range(nh):
        for p in range(NPCOL):
            run_pass(
                table,
                lambda b, hh=hh, p=p: srcidx.at[c, hh, p, pl.ds(b, CHUNK)],
                lambda b, hh=hh: dstidx.at[c, hh, pl.ds(b, CHUNK)],
                s_out.at[c, hh, p, pl.ds(rbase, rpt)],
            )
        if with_deg:
            run_pass(
                ones_tab,
                lambda b: degidx.at[pl.ds(b, CHUNK)],
                lambda b, hh=hh: dstidx.at[c, hh, pl.ds(b, CHUNK)],
                deg_out.at[c, hh, pl.ds(rbase, rpt)],
            )


@functools.cache
def _make_seg_kernel(nr_acc, nh, with_deg):
    out_type = [jax.ShapeDtypeStruct((NCORES, nh, NPCOL, nr_acc, W), jnp.float32)]
    if with_deg:
        out_type.append(
            jax.ShapeDtypeStruct((NCORES, nh, nr_acc, W), jnp.float32))
    scratch = (
        pltpu.VMEM_SHARED((nr_acc, W), jnp.float32),
        pltpu.VMEM((CHUNK,), jnp.int32),
        pltpu.VMEM((CHUNK,), jnp.int32),
        pltpu.VMEM((CHUNK, W), jnp.float32),
        pltpu.SemaphoreType.DMA,
    )
    mesh = plsc.VectorSubcoreMesh(core_axis_name="c", subcore_axis_name="s")
    body = functools.partial(_seg_body, nr_acc, nh, with_deg)
    return pl.kernel(body, out_type=tuple(out_type), mesh=mesh,
                     scratch_types=scratch,
                     name=f"segsum_{nr_acc}x{nh}" + ("_deg" if with_deg else ""))


def _combine_w(coeff, basis):
    def body(coeff_ref, basis_ref, w_ref):
        for r in range(NR):
            acc = coeff_ref[r, 0] * basis_ref[0]
            for b in range(1, NB):
                acc = acc + coeff_ref[r, b] * basis_ref[b]
            w_ref[r] = acc

    return pl.pallas_call(
        body,
        in_specs=[pl.BlockSpec(memory_space=pltpu.SMEM),
                  pl.BlockSpec((NB, FEAT, FEAT), lambda: (0, 0, 0))],
        out_specs=pl.BlockSpec((NR, FEAT, FEAT), lambda: (0, 0, 0)),
        out_shape=jax.ShapeDtypeStruct((NR, FEAT, FEAT), jnp.float32),
    )(coeff, basis)


def _agg_block(s_ref, deg_ref, wp_ref):
    acc = None
    for a in range(2):
        d = deg_ref[a, 0][:, 0:1]
        inv = 1.0 / jnp.maximum(d, 1.0)
        sa = jnp.concatenate([s_ref[a, 0, p] for p in range(NPCOL)], axis=1)
        t = jnp.dot(sa * inv, wp_ref[a], preferred_element_type=jnp.float32)
        acc = t if acc is None else acc + t
    return acc


def _tc_layer(h, s, deg, wp, loop_w, bias, dec, n_half, r_blk, hh, row0):
    grid = n_half // r_blk
    blk0 = row0 // r_blk

    def body(h_ref, s_ref, deg_ref, wp_ref, loop_ref, bias_ref, *rest):
        o_ref = rest[-1]
        acc = jnp.dot(h_ref[...], loop_ref[...],
                      preferred_element_type=jnp.float32)
        acc = acc + _agg_block(s_ref, deg_ref, wp_ref)
        x = jnp.maximum(acc + bias_ref[...], 0.0)
        if dec is None:
            o_ref[...] = x
        else:
            dw1_ref, db1_ref, dw2_ref, db2_ref = rest[:-1]
            y = jnp.maximum(
                jnp.dot(x, dw1_ref[...], preferred_element_type=jnp.float32)
                + db1_ref[...], 0.0)
            o_ref[...] = (jnp.dot(y, dw2_ref[...],
                                  preferred_element_type=jnp.float32)
                          + db2_ref[...])

    in_specs = [
        pl.BlockSpec((r_blk, FEAT), lambda i: (i + blk0, 0)),
        pl.BlockSpec((2, 1, NPCOL, r_blk, W), lambda i: (0, hh, 0, i, 0)),
        pl.BlockSpec((2, 1, r_blk, W), lambda i: (0, hh, i, 0)),
        pl.BlockSpec((2, FEAT, FEAT), lambda i: (0, 0, 0)),
        pl.BlockSpec((FEAT, FEAT), lambda i: (0, 0)),
        pl.BlockSpec((1, FEAT), lambda i: (0, 0)),
    ]
    args = [h, s, deg, wp, loop_w, bias]
    out_w = FEAT
    if dec is not None:
        dw1, db1, dw2, db2 = dec
        in_specs += [
            pl.BlockSpec((FEAT, FEAT), lambda i: (0, 0)),
            pl.BlockSpec((1, FEAT), lambda i: (0, 0)),
            pl.BlockSpec((FEAT, OUT), lambda i: (0, 0)),
            pl.BlockSpec((1, OUT), lambda i: (0, 0)),
        ]
        args += [dw1, db1, dw2, db2]
        out_w = OUT

    return pl.pallas_call(
        body,
        grid=(grid,),
        in_specs=in_specs,
        out_specs=pl.BlockSpec((r_blk, out_w), lambda i: (i, 0)),
        out_shape=jax.ShapeDtypeStruct((n_half, out_w), jnp.float32),
    )(*args)


def kernel(h_lnc, h_mi, h_m, src0, dst0, src1, dst1, src2, dst2, src3, dst3,
           src4, dst4, src5, dst5, basis1, coeff1, loop1, bias1, basis2,
           coeff2, loop2, bias2, dec_W1, dec_b1, dec_W2, dec_b2):
    srcs = [src0, src1, src2, src3, src4, src5]
    dsts = [dst0, dst1, dst2, dst3, dst4, dst5]

    W1s = _combine_w(coeff1, basis1)
    W2s = _combine_w(coeff2, basis2)

    epad = E_PAD - E
    eidx = jnp.arange(E_PAD, dtype=jnp.int32)
    spread = eidx % ONES_ROWS
    grp_idx = {}
    for g in GROUPS:
        nh, nr_acc = g["nh"], g["nr_acc"]
        src_all, dst_all = [], []
        for r, st in zip(g["rels"], g["srct"]):
            sp = jnp.concatenate(
                [srcs[r] + OFF[st], jnp.zeros((epad,), jnp.int32)])
            dp = jnp.concatenate(
                [dsts[r], jnp.full((epad,), jnp.int32(1 << 28))])
            src_h, dst_h = [], []
            for hh in range(nh):
                lo = hh * H_SPLIT_M if nh > 1 else 0
                hi = lo + g["h_sz"][hh] if nh > 1 else g["n"]
                ok = (dp >= lo) & (dp < hi)
                trash = (nr_acc - 16) + (eidx & 15)
                dst_h.append(jnp.where(ok, dp - lo, trash))
                src_h.append(jnp.where(ok, sp, spread) * NPCOL)
            src_all.append(jnp.stack(src_h))
            dst_all.append(jnp.stack(dst_h))
        src_base = jnp.stack(src_all)
        grp_idx[g["name"]] = (
            jnp.stack([src_base + p for p in range(NPCOL)], axis=2),
            jnp.stack(dst_all),
        )

    degidx = spread * NPCOL
    ones_tab = jnp.ones((ONES_ROWS * NPCOL, W), jnp.float32)
    zeros_big = jnp.zeros((808 * W,), jnp.float32)

    h_parts = [h_lnc, h_mi, h_m]
    degs = {}
    final = []
    for layer in (0, 1):
        Ws = W1s if layer == 0 else W2s
        h_all = jnp.concatenate(h_parts, axis=0)
        table = h_all.reshape(N_TOT * NPCOL, W)
        s_outs = {}
        for g in GROUPS:
            name, nh, nr_acc = g["name"], g["nh"], g["nr_acc"]
            rpt = nr_acc // NS
            srcidx, dstidx = grp_idx[name]
            zeros_w = zeros_big[: rpt * W].reshape(rpt, W)
            kern = _make_seg_kernel(nr_acc, nh, layer == 0)
            if layer == 0:
                s_out, deg = kern(table, ones_tab, srcidx, dstidx, degidx,
                                  zeros_w)
                degs[name] = deg
            else:
                (s_out,) = kern(table, srcidx, dstidx, zeros_w)
            s_outs[name] = s_out

        loop_w = loop1 if layer == 0 else loop2
        bias = (bias1 if layer == 0 else bias2).reshape(1, FEAT)
        dec = (None if layer == 0 else
               (dec_W1, dec_b1.reshape(1, FEAT), dec_W2, dec_b2.reshape(1, OUT)))
        new_h = []
        hpos = 0
        for gi, g in enumerate(GROUPS):
            name = g["name"]
            wp = jnp.stack([Ws[g["rels"][0]], Ws[g["rels"][1]]])
            row0 = 0
            for hh in range(g["nh"]):
                o = _tc_layer(h_parts[gi], s_outs[name], degs[name], wp,
                              loop_w, bias, dec, g["h_sz"][hh], g["r_blk"],
                              hh, row0)
                row0 += g["h_sz"][hh]
                if layer == 0:
                    new_h.append(o)
                else:
                    final.append(o)
        if layer == 0:
            h_parts = [new_h[0], new_h[1],
                       jnp.concatenate(new_h[2:], axis=0)]

    return jnp.concatenate(final, axis=0)


# speedup vs baseline: 5.3253x; 2.3982x over previous
import functools

import jax
import jax.numpy as jnp
from jax import lax
from jax.experimental import pallas as pl
from jax.experimental.pallas import tpu as pltpu
from jax.experimental.pallas import tpu_sc as plsc

N_LNC, N_MI, N_M = 10000, 5000, 20000
N_TOT = N_LNC + N_MI + N_M
OFF = {"lnc": 0, "mi": N_LNC, "m": N_LNC + N_MI}
FEAT = 256
OUT = 128
E = 50000
NB = 4
NR = 6

NCORES = 2
NS = 16
W = 128
NPCOL = FEAT // W
EPT = 3200
E_PAD = NS * EPT
ONES_ROWS = 2048

GROUPS = (
    dict(name="lnc", n=N_LNC, nh=1, h_sz=(N_LNC,), nr_acc=10112, r_blk=1000,
         chunk=128, rels=(1, 5), srct=("mi", "m")),
    dict(name="mi", n=N_MI, nh=1, h_sz=(N_MI,), nr_acc=5120, r_blk=1000,
         chunk=128, rels=(0, 3), srct=("lnc", "m")),
    dict(name="m", n=N_M, nh=2, h_sz=(10000, 10000), nr_acc=10112, r_blk=400,
         chunk=128, rels=(2, 4), srct=("mi", "lnc")),
)
H_SPLIT_M = 10000


def _seg_body(nr_acc, nh, chunk, with_deg, *refs):
    rpt = nr_acc // NS
    nchunk = EPT // chunk
    if with_deg:
        (table, ones_tab, srcidx, dstidx, degidx, zeros_hbm,
         s_out, deg_out, acc, srci_all, dsti_all, rows0, rows1,
         gsem0, gsem1) = refs
    else:
        (table, srcidx, dstidx, zeros_hbm,
         s_out, acc, srci_all, dsti_all, rows0, rows1, gsem0, gsem1) = refs
    c = lax.axis_index("c")
    s = lax.axis_index("s")
    rbase = s * rpt

    def run_pass(tab, src_slab, dst_slab, out_at):
        pltpu.sync_copy(zeros_hbm, acc.at[pl.ds(rbase, rpt)])
        pltpu.sync_copy(src_slab, srci_all)
        pltpu.sync_copy(dst_slab, dsti_all)
        plsc.subcore_barrier()

        def issue(j, rows, sem):
            pltpu.async_copy(tab.at[srci_all.at[j]], rows, sem)

        def wait_scat(j, rows, sem):
            pltpu.make_async_copy(tab.at[srci_all.at[j]], rows, sem).wait()
            pltpu.sync_copy(rows, acc.at[dsti_all.at[j]], add=True)

        issue(0, rows0, gsem0)
        npairs = (nchunk - 1) // 2 if nchunk % 2 else (nchunk - 2) // 2

        def pbody(t, carry):
            j0 = 2 * t
            issue(j0 + 1, rows1, gsem1)
            wait_scat(j0, rows0, gsem0)
            issue(j0 + 2, rows0, gsem0)
            wait_scat(j0 + 1, rows1, gsem1)
            return carry

        lax.fori_loop(0, npairs, pbody, 0)
        if nchunk % 2:
            wait_scat(nchunk - 1, rows0, gsem0)
        else:
            issue(nchunk - 1, rows1, gsem1)
            wait_scat(nchunk - 2, rows0, gsem0)
            wait_scat(nchunk - 1, rows1, gsem1)
        plsc.subcore_barrier()
        pltpu.sync_copy(acc.at[pl.ds(rbase, rpt)], out_at)

    for hh in range(nh):
        for p in range(NPCOL):
            run_pass(
                table,
                srcidx.at[c, hh, p, s],
                dstidx.at[c, hh, s],
                s_out.at[c, hh, p, pl.ds(rbase, rpt)],
            )
        if with_deg:
            run_pass(
                ones_tab,
                degidx.at[s],
                dstidx.at[c, hh, s],
                deg_out.at[c, hh, pl.ds(rbase, rpt)],
            )


@functools.cache
def _make_seg_kernel(nr_acc, nh, chunk, with_deg):
    nchunk = EPT // chunk
    out_type = [jax.ShapeDtypeStruct((NCORES, nh, NPCOL, nr_acc, W), jnp.float32)]
    if with_deg:
        out_type.append(
            jax.ShapeDtypeStruct((NCORES, nh, nr_acc, W), jnp.float32))
    scratch = (
        pltpu.VMEM_SHARED((nr_acc, W), jnp.float32),
        pltpu.VMEM((nchunk, chunk), jnp.int32),
        pltpu.VMEM((nchunk, chunk), jnp.int32),
        pltpu.VMEM((chunk, W), jnp.float32),
        pltpu.VMEM((chunk, W), jnp.float32),
        pltpu.SemaphoreType.DMA,
        pltpu.SemaphoreType.DMA,
    )
    mesh = plsc.VectorSubcoreMesh(core_axis_name="c", subcore_axis_name="s")
    body = functools.partial(_seg_body, nr_acc, nh, chunk, with_deg)
    return pl.kernel(body, out_type=tuple(out_type), mesh=mesh,
                     scratch_types=scratch,
                     name=f"segsum_{nr_acc}x{nh}" + ("_deg" if with_deg else ""))


def _combine_w(coeff, basis):
    def body(coeff_ref, basis_ref, w_ref):
        for r in range(NR):
            acc = coeff_ref[r, 0] * basis_ref[0]
            for b in range(1, NB):
                acc = acc + coeff_ref[r, b] * basis_ref[b]
            w_ref[r] = acc

    return pl.pallas_call(
        body,
        in_specs=[pl.BlockSpec(memory_space=pltpu.SMEM),
                  pl.BlockSpec((NB, FEAT, FEAT), lambda: (0, 0, 0))],
        out_specs=pl.BlockSpec((NR, FEAT, FEAT), lambda: (0, 0, 0)),
        out_shape=jax.ShapeDtypeStruct((NR, FEAT, FEAT), jnp.float32),
    )(coeff, basis)


def _agg_block(s_ref, deg_ref, wp_ref):
    acc = None
    for a in range(2):
        d = deg_ref[a, 0][:, 0:1]
        inv = 1.0 / jnp.maximum(d, 1.0)
        sa = jnp.concatenate([s_ref[a, 0, p] for p in range(NPCOL)], axis=1)
        t = jnp.dot(sa * inv, wp_ref[a], preferred_element_type=jnp.float32)
        acc = t if acc is None else acc + t
    return acc


def _tc_layer(h, s, deg, wp, loop_w, bias, dec, n_half, r_blk, hh, row0):
    grid = n_half // r_blk
    blk0 = row0 // r_blk

    def body(h_ref, s_ref, deg_ref, wp_ref, loop_ref, bias_ref, *rest):
        o_ref = rest[-1]
        acc = jnp.dot(h_ref[...], loop_ref[...],
                      preferred_element_type=jnp.float32)
        acc = acc + _agg_block(s_ref, deg_ref, wp_ref)
        x = jnp.maximum(acc + bias_ref[...], 0.0)
        if dec is None:
            o_ref[...] = x
        else:
            dw1_ref, db1_ref, dw2_ref, db2_ref = rest[:-1]
            y = jnp.maximum(
                jnp.dot(x, dw1_ref[...], preferred_element_type=jnp.float32)
                + db1_ref[...], 0.0)
            o_ref[...] = (jnp.dot(y, dw2_ref[...],
                                  preferred_element_type=jnp.float32)
                          + db2_ref[...])

    in_specs = [
        pl.BlockSpec((r_blk, FEAT), lambda i: (i + blk0, 0)),
        pl.BlockSpec((2, 1, NPCOL, r_blk, W), lambda i: (0, hh, 0, i, 0)),
        pl.BlockSpec((2, 1, r_blk, W), lambda i: (0, hh, i, 0)),
        pl.BlockSpec((2, FEAT, FEAT), lambda i: (0, 0, 0)),
        pl.BlockSpec((FEAT, FEAT), lambda i: (0, 0)),
        pl.BlockSpec((1, FEAT), lambda i: (0, 0)),
    ]
    args = [h, s, deg, wp, loop_w, bias]
    out_w = FEAT
    if dec is not None:
        dw1, db1, dw2, db2 = dec
        in_specs += [
            pl.BlockSpec((FEAT, FEAT), lambda i: (0, 0)),
            pl.BlockSpec((1, FEAT), lambda i: (0, 0)),
            pl.BlockSpec((FEAT, OUT), lambda i: (0, 0)),
            pl.BlockSpec((1, OUT), lambda i: (0, 0)),
        ]
        args += [dw1, db1, dw2, db2]
        out_w = OUT

    return pl.pallas_call(
        body,
        grid=(grid,),
        in_specs=in_specs,
        out_specs=pl.BlockSpec((r_blk, out_w), lambda i: (i, 0)),
        out_shape=jax.ShapeDtypeStruct((n_half, out_w), jnp.float32),
    )(*args)


def kernel(h_lnc, h_mi, h_m, src0, dst0, src1, dst1, src2, dst2, src3, dst3,
           src4, dst4, src5, dst5, basis1, coeff1, loop1, bias1, basis2,
           coeff2, loop2, bias2, dec_W1, dec_b1, dec_W2, dec_b2):
    srcs = [src0, src1, src2, src3, src4, src5]
    dsts = [dst0, dst1, dst2, dst3, dst4, dst5]

    W1s = _combine_w(coeff1, basis1)
    W2s = _combine_w(coeff2, basis2)

    epad = E_PAD - E
    eidx = jnp.arange(E_PAD, dtype=jnp.int32)
    spread = eidx % ONES_ROWS
    grp_idx = {}
    for g in GROUPS:
        nh, nr_acc = g["nh"], g["nr_acc"]
        src_all, dst_all = [], []
        for r, st in zip(g["rels"], g["srct"]):
            sp = jnp.concatenate(
                [srcs[r] + OFF[st], jnp.zeros((epad,), jnp.int32)])
            dp = jnp.concatenate(
                [dsts[r], jnp.full((epad,), jnp.int32(1 << 28))])
            src_h, dst_h = [], []
            for hh in range(nh):
                lo = hh * H_SPLIT_M if nh > 1 else 0
                hi = lo + g["h_sz"][hh] if nh > 1 else g["n"]
                ok = (dp >= lo) & (dp < hi)
                trash = (nr_acc - 16) + (eidx & 15)
                dst_h.append(jnp.where(ok, dp - lo, trash))
                src_h.append(jnp.where(ok, sp, spread) * NPCOL)
            src_all.append(jnp.stack(src_h))
            dst_all.append(jnp.stack(dst_h))
        ck = g["chunk"]
        nck = EPT // ck
        src_base = jnp.stack(src_all)
        grp_idx[g["name"]] = (
            jnp.stack([src_base + p for p in range(NPCOL)], axis=2)
            .reshape(NCORES, nh, NPCOL, NS, nck, ck),
            jnp.stack(dst_all).reshape(NCORES, nh, NS, nck, ck),
        )

    degidx = spread * NPCOL
    ones_tab = jnp.ones((ONES_ROWS * NPCOL, W), jnp.float32)
    zeros_big = jnp.zeros((808 * W,), jnp.float32)

    h_parts = [h_lnc, h_mi, h_m]
    degs = {}
    final = []
    for layer in (0, 1):
        Ws = W1s if layer == 0 else W2s
        h_all = jnp.concatenate(h_parts, axis=0)
        table = h_all.reshape(N_TOT * NPCOL, W)
        s_outs = {}
        for g in GROUPS:
            name, nh, nr_acc, ck = g["name"], g["nh"], g["nr_acc"], g["chunk"]
            rpt = nr_acc // NS
            srcidx, dstidx = grp_idx[name]
            zeros_w = zeros_big[: rpt * W].reshape(rpt, W)
            kern = _make_seg_kernel(nr_acc, nh, ck, layer == 0)
            if layer == 0:
                degidx_g = degidx.reshape(NS, EPT // ck, ck)
                s_out, deg = kern(table, ones_tab, srcidx, dstidx, degidx_g,
                                  zeros_w)
                degs[name] = deg
            else:
                (s_out,) = kern(table, srcidx, dstidx, zeros_w)
            s_outs[name] = s_out

        loop_w = loop1 if layer == 0 else loop2
        bias = (bias1 if layer == 0 else bias2).reshape(1, FEAT)
        dec = (None if layer == 0 else
               (dec_W1, dec_b1.reshape(1, FEAT), dec_W2, dec_b2.reshape(1, OUT)))
        new_h = []
        hpos = 0
        for gi, g in enumerate(GROUPS):
            name = g["name"]
            wp = jnp.stack([Ws[g["rels"][0]], Ws[g["rels"][1]]])
            row0 = 0
            for hh in range(g["nh"]):
                o = _tc_layer(h_parts[gi], s_outs[name], degs[name], wp,
                              loop_w, bias, dec, g["h_sz"][hh], g["r_blk"],
                              hh, row0)
                row0 += g["h_sz"][hh]
                if layer == 0:
                    new_h.append(o)
                else:
                    final.append(o)
        if layer == 0:
            h_parts = [new_h[0], new_h[1],
                       jnp.concatenate(new_h[2:], axis=0)]

    return jnp.concatenate(final, axis=0)


# speedup vs baseline: 5.5555x; 1.0432x over previous
import functools

import jax
import jax.numpy as jnp
from jax import lax
from jax.experimental import pallas as pl
from jax.experimental.pallas import tpu as pltpu
from jax.experimental.pallas import tpu_sc as plsc

N_LNC, N_MI, N_M = 10000, 5000, 20000
N_TOT = N_LNC + N_MI + N_M
OFF = {"lnc": 0, "mi": N_LNC, "m": N_LNC + N_MI}
FEAT = 256
OUT = 128
E = 50000
NB = 4
NR = 6

NCORES = 2
NS = 16
W = 128
NPCOL = FEAT // W
EPT = 3200
E_PAD = NS * EPT
ONES_ROWS = 2048

GROUPS = (
    dict(name="lnc", n=N_LNC, nh=1, h_sz=(N_LNC,), nr_acc=10112, r_blk=1000,
         chunk=128, rels=(1, 5), srct=("mi", "m")),
    dict(name="mi", n=N_MI, nh=1, h_sz=(N_MI,), nr_acc=5120, r_blk=1000,
         chunk=128, rels=(0, 3), srct=("lnc", "m")),
    dict(name="m", n=N_M, nh=2, h_sz=(10000, 10000), nr_acc=10112, r_blk=400,
         chunk=128, rels=(2, 4), srct=("mi", "lnc")),
)
H_SPLIT_M = 10000


def _seg_body(nr_acc, nh, chunk, with_deg, *refs):
    rpt = nr_acc // NS
    nchunk = EPT // chunk
    if with_deg:
        (table, ones_hbm, srcidx, dstidx, zeros_hbm,
         s_out, deg_out, acc, srci_all, dsti_all, rows0, rows1,
         gsem0, gsem1, zsem) = refs
    else:
        (table, srcidx, dstidx, zeros_hbm,
         s_out, acc, srci_all, dsti_all, rows0, rows1, gsem0, gsem1,
         zsem) = refs
    c = lax.axis_index("c")
    s = lax.axis_index("s")
    rbase = s * rpt

    def zero_acc():
        pltpu.async_copy(zeros_hbm, acc.at[pl.ds(rbase, rpt)], zsem)

    def zero_wait():
        pltpu.make_async_copy(zeros_hbm, acc.at[pl.ds(rbase, rpt)],
                              zsem).wait()

    def run_pass(tab, src_slab, dst_slab, out_at):
        zero_acc()
        pltpu.sync_copy(src_slab, srci_all)
        pltpu.sync_copy(dst_slab, dsti_all)
        zero_wait()
        plsc.subcore_barrier()

        def issue(j, rows, sem):
            pltpu.async_copy(tab.at[srci_all.at[j]], rows, sem)

        def wait_scat(j, rows, sem):
            pltpu.make_async_copy(tab.at[srci_all.at[j]], rows, sem).wait()
            pltpu.sync_copy(rows, acc.at[dsti_all.at[j]], add=True)

        issue(0, rows0, gsem0)
        npairs = (nchunk - 1) // 2 if nchunk % 2 else (nchunk - 2) // 2

        def pbody(t, carry):
            j0 = 2 * t
            issue(j0 + 1, rows1, gsem1)
            wait_scat(j0, rows0, gsem0)
            issue(j0 + 2, rows0, gsem0)
            wait_scat(j0 + 1, rows1, gsem1)
            return carry

        lax.fori_loop(0, npairs, pbody, 0)
        if nchunk % 2:
            wait_scat(nchunk - 1, rows0, gsem0)
        else:
            issue(nchunk - 1, rows1, gsem1)
            wait_scat(nchunk - 2, rows0, gsem0)
            wait_scat(nchunk - 1, rows1, gsem1)
        plsc.subcore_barrier()
        pltpu.sync_copy(acc.at[pl.ds(rbase, rpt)], out_at)

    def run_deg_pass(dst_slab, out_at):
        zero_acc()
        pltpu.sync_copy(dst_slab, dsti_all)
        pltpu.sync_copy(ones_hbm, rows0)
        zero_wait()
        plsc.subcore_barrier()

        def dbody(j, carry):
            pltpu.sync_copy(rows0, acc.at[dsti_all.at[j]], add=True)
            return carry

        lax.fori_loop(0, EPT // chunk, dbody, 0)
        plsc.subcore_barrier()
        pltpu.sync_copy(acc.at[pl.ds(rbase, rpt)], out_at)

    for hh in range(nh):
        for p in range(NPCOL):
            run_pass(
                table,
                srcidx.at[c, hh, p, s],
                dstidx.at[c, hh, s],
                s_out.at[c, hh, p, pl.ds(rbase, rpt)],
            )
        if with_deg:
            run_deg_pass(
                dstidx.at[c, hh, s],
                deg_out.at[c, hh, pl.ds(rbase, rpt)],
            )


@functools.cache
def _make_seg_kernel(nr_acc, nh, chunk, with_deg):
    nchunk = EPT // chunk
    out_type = [jax.ShapeDtypeStruct((NCORES, nh, NPCOL, nr_acc, W), jnp.float32)]
    if with_deg:
        out_type.append(
            jax.ShapeDtypeStruct((NCORES, nh, nr_acc, W), jnp.float32))
    scratch = (
        pltpu.VMEM_SHARED((nr_acc, W), jnp.float32),
        pltpu.VMEM((nchunk, chunk), jnp.int32),
        pltpu.VMEM((nchunk, chunk), jnp.int32),
        pltpu.VMEM((chunk, W), jnp.float32),
        pltpu.VMEM((chunk, W), jnp.float32),
        pltpu.SemaphoreType.DMA,
        pltpu.SemaphoreType.DMA,
        pltpu.SemaphoreType.DMA,
    )
    mesh = plsc.VectorSubcoreMesh(core_axis_name="c", subcore_axis_name="s")
    body = functools.partial(_seg_body, nr_acc, nh, chunk, with_deg)
    return pl.kernel(body, out_type=tuple(out_type), mesh=mesh,
                     scratch_types=scratch,
                     name=f"segsum_{nr_acc}x{nh}" + ("_deg" if with_deg else ""))


def _combine_w(coeff, basis):
    def body(coeff_ref, basis_ref, w_ref):
        for r in range(NR):
            acc = coeff_ref[r, 0] * basis_ref[0]
            for b in range(1, NB):
                acc = acc + coeff_ref[r, b] * basis_ref[b]
            w_ref[r] = acc

    return pl.pallas_call(
        body,
        in_specs=[pl.BlockSpec(memory_space=pltpu.SMEM),
                  pl.BlockSpec((NB, FEAT, FEAT), lambda: (0, 0, 0))],
        out_specs=pl.BlockSpec((NR, FEAT, FEAT), lambda: (0, 0, 0)),
        out_shape=jax.ShapeDtypeStruct((NR, FEAT, FEAT), jnp.float32),
    )(coeff, basis)


def _agg_block(s_ref, deg_ref, wp_ref):
    acc = None
    for a in range(2):
        d = deg_ref[a, 0][:, 0:1]
        inv = 1.0 / jnp.maximum(d, 1.0)
        sa = jnp.concatenate([s_ref[a, 0, p] for p in range(NPCOL)], axis=1)
        t = jnp.dot(sa * inv, wp_ref[a], preferred_element_type=jnp.float32)
        acc = t if acc is None else acc + t
    return acc


def _tc_layer(h, s, deg, wp, loop_w, bias, dec, n_half, r_blk, hh, row0):
    grid = n_half // r_blk
    blk0 = row0 // r_blk

    def body(h_ref, s_ref, deg_ref, wp_ref, loop_ref, bias_ref, *rest):
        o_ref = rest[-1]
        acc = jnp.dot(h_ref[...], loop_ref[...],
                      preferred_element_type=jnp.float32)
        acc = acc + _agg_block(s_ref, deg_ref, wp_ref)
        x = jnp.maximum(acc + bias_ref[...], 0.0)
        if dec is None:
            o_ref[...] = x
        else:
            dw1_ref, db1_ref, dw2_ref, db2_ref = rest[:-1]
            y = jnp.maximum(
                jnp.dot(x, dw1_ref[...], preferred_element_type=jnp.float32)
                + db1_ref[...], 0.0)
            o_ref[...] = (jnp.dot(y, dw2_ref[...],
                                  preferred_element_type=jnp.float32)
                          + db2_ref[...])

    in_specs = [
        pl.BlockSpec((r_blk, FEAT), lambda i: (i + blk0, 0)),
        pl.BlockSpec((2, 1, NPCOL, r_blk, W), lambda i: (0, hh, 0, i, 0)),
        pl.BlockSpec((2, 1, r_blk, W), lambda i: (0, hh, i, 0)),
        pl.BlockSpec((2, FEAT, FEAT), lambda i: (0, 0, 0)),
        pl.BlockSpec((FEAT, FEAT), lambda i: (0, 0)),
        pl.BlockSpec((1, FEAT), lambda i: (0, 0)),
    ]
    args = [h, s, deg, wp, loop_w, bias]
    out_w = FEAT
    if dec is not None:
        dw1, db1, dw2, db2 = dec
        in_specs += [
            pl.BlockSpec((FEAT, FEAT), lambda i: (0, 0)),
            pl.BlockSpec((1, FEAT), lambda i: (0, 0)),
            pl.BlockSpec((FEAT, OUT), lambda i: (0, 0)),
            pl.BlockSpec((1, OUT), lambda i: (0, 0)),
        ]
        args += [dw1, db1, dw2, db2]
        out_w = OUT

    return pl.pallas_call(
        body,
        grid=(grid,),
        in_specs=in_specs,
        out_specs=pl.BlockSpec((r_blk, out_w), lambda i: (i, 0)),
        out_shape=jax.ShapeDtypeStruct((n_half, out_w), jnp.float32),
    )(*args)


def kernel(h_lnc, h_mi, h_m, src0, dst0, src1, dst1, src2, dst2, src3, dst3,
           src4, dst4, src5, dst5, basis1, coeff1, loop1, bias1, basis2,
           coeff2, loop2, bias2, dec_W1, dec_b1, dec_W2, dec_b2):
    srcs = [src0, src1, src2, src3, src4, src5]
    dsts = [dst0, dst1, dst2, dst3, dst4, dst5]

    W1s = _combine_w(coeff1, basis1)
    W2s = _combine_w(coeff2, basis2)

    epad = E_PAD - E
    eidx = jnp.arange(E_PAD, dtype=jnp.int32)
    spread = eidx % ONES_ROWS
    grp_idx = {}
    for g in GROUPS:
        nh, nr_acc = g["nh"], g["nr_acc"]
        src_all, dst_all = [], []
        for r, st in zip(g["rels"], g["srct"]):
            sp = jnp.concatenate(
                [srcs[r] + OFF[st], jnp.zeros((epad,), jnp.int32)])
            dp = jnp.concatenate(
                [dsts[r], jnp.full((epad,), jnp.int32(1 << 28))])
            src_h, dst_h = [], []
            for hh in range(nh):
                lo = hh * H_SPLIT_M if nh > 1 else 0
                hi = lo + g["h_sz"][hh] if nh > 1 else g["n"]
                ok = (dp >= lo) & (dp < hi)
                trash = (nr_acc - 16) + (eidx & 15)
                dst_h.append(jnp.where(ok, dp - lo, trash))
                src_h.append(jnp.where(ok, sp, spread) * NPCOL)
            src_all.append(jnp.stack(src_h))
            dst_all.append(jnp.stack(dst_h))
        ck = g["chunk"]
        nck = EPT // ck
        src_base = jnp.stack(src_all)
        grp_idx[g["name"]] = (
            jnp.stack([src_base + p for p in range(NPCOL)], axis=2)
            .reshape(NCORES, nh, NPCOL, NS, nck, ck),
            jnp.stack(dst_all).reshape(NCORES, nh, NS, nck, ck),
        )

    ones_tab = jnp.ones((128, W), jnp.float32)
    zeros_big = jnp.zeros((808 * W,), jnp.float32)

    h_parts = [h_lnc, h_mi, h_m]
    degs = {}
    final = []
    for layer in (0, 1):
        Ws = W1s if layer == 0 else W2s
        h_all = jnp.concatenate(h_parts, axis=0)
        table = h_all.reshape(N_TOT * NPCOL, W)
        s_outs = {}
        for g in GROUPS:
            name, nh, nr_acc, ck = g["name"], g["nh"], g["nr_acc"], g["chunk"]
            rpt = nr_acc // NS
            srcidx, dstidx = grp_idx[name]
            zeros_w = zeros_big[: rpt * W].reshape(rpt, W)
            kern = _make_seg_kernel(nr_acc, nh, ck, layer == 0)
            if layer == 0:
                s_out, deg = kern(table, ones_tab, srcidx, dstidx, zeros_w)
                degs[name] = deg
            else:
                (s_out,) = kern(table, srcidx, dstidx, zeros_w)
            s_outs[name] = s_out

        loop_w = loop1 if layer == 0 else loop2
        bias = (bias1 if layer == 0 else bias2).reshape(1, FEAT)
        dec = (None if layer == 0 else
               (dec_W1, dec_b1.reshape(1, FEAT), dec_W2, dec_b2.reshape(1, OUT)))
        new_h = []
        hpos = 0
        for gi, g in enumerate(GROUPS):
            name = g["name"]
            wp = jnp.stack([Ws[g["rels"][0]], Ws[g["rels"][1]]])
            row0 = 0
            for hh in range(g["nh"]):
                o = _tc_layer(h_parts[gi], s_outs[name], degs[name], wp,
                              loop_w, bias, dec, g["h_sz"][hh], g["r_blk"],
                              hh, row0)
                row0 += g["h_sz"][hh]
                if layer == 0:
                    new_h.append(o)
                else:
                    final.append(o)
        if layer == 0:
            h_parts = [new_h[0], new_h[1],
                       jnp.concatenate(new_h[2:], axis=0)]

    return jnp.concatenate(final, axis=0)


# speedup vs baseline: 5.7401x; 1.0332x over previous
import functools

import jax
import jax.numpy as jnp
from jax import lax
from jax.experimental import pallas as pl
from jax.experimental.pallas import tpu as pltpu
from jax.experimental.pallas import tpu_sc as plsc

N_LNC, N_MI, N_M = 10000, 5000, 20000
N_TOT = N_LNC + N_MI + N_M
OFF = {"lnc": 0, "mi": N_LNC, "m": N_LNC + N_MI}
FEAT = 256
OUT = 128
E = 50000
NB = 4
NR = 6

NCORES = 2
NS = 16
W = 128
NPCOL = FEAT // W
CHUNK = 128
EPT = 3200
E_PAD = NS * EPT
NCHUNK = EPT // CHUNK
NRACC = 10112
RPT = NRACC // NS
NSEG = 4
SEGMENTS = (
    dict(rels=(1, 5), srct=("mi", "m"), dst_lo=0, n=N_LNC),
    dict(rels=(0, 3), srct=("lnc", "m"), dst_lo=0, n=N_MI),
    dict(rels=(2, 4), srct=("mi", "lnc"), dst_lo=0, n=10000),
    dict(rels=(2, 4), srct=("mi", "lnc"), dst_lo=10000, n=10000),
)
R_BLK = 1000
SEG_STARTS = (10, 15, 25)


def _seg_body(with_deg, *refs):
    if with_deg:
        (table, ones_hbm, srcidx, dstidx, zeros_hbm,
         s_out, deg_out, acc, srci_all, dsti_all, rows0, rows1,
         gsem0, gsem1, zsem) = refs
    else:
        (table, srcidx, dstidx, zeros_hbm,
         s_out, acc, srci_all, dsti_all, rows0, rows1, gsem0, gsem1,
         zsem) = refs
    c = lax.axis_index("c")
    s = lax.axis_index("s")
    rbase = s * RPT

    def zero_acc():
        pltpu.async_copy(zeros_hbm, acc.at[pl.ds(rbase, RPT)], zsem)

    def zero_wait():
        pltpu.make_async_copy(zeros_hbm, acc.at[pl.ds(rbase, RPT)],
                              zsem).wait()

    def run_pass(src_slab, dst_slab, out_at):
        zero_acc()
        pltpu.sync_copy(src_slab, srci_all)
        pltpu.sync_copy(dst_slab, dsti_all)
        zero_wait()
        plsc.subcore_barrier()

        def issue(j, rows, sem):
            pltpu.async_copy(table.at[srci_all.at[j]], rows, sem)

        def wait_scat(j, rows, sem):
            pltpu.make_async_copy(table.at[srci_all.at[j]], rows, sem).wait()
            pltpu.sync_copy(rows, acc.at[dsti_all.at[j]], add=True)

        issue(0, rows0, gsem0)
        npairs = (NCHUNK - 1) // 2 if NCHUNK % 2 else (NCHUNK - 2) // 2

        def pbody(t, carry):
            j0 = 2 * t
            issue(j0 + 1, rows1, gsem1)
            wait_scat(j0, rows0, gsem0)
            issue(j0 + 2, rows0, gsem0)
            wait_scat(j0 + 1, rows1, gsem1)
            return carry

        lax.fori_loop(0, npairs, pbody, 0)
        if NCHUNK % 2:
            wait_scat(NCHUNK - 1, rows0, gsem0)
        else:
            issue(NCHUNK - 1, rows1, gsem1)
            wait_scat(NCHUNK - 2, rows0, gsem0)
            wait_scat(NCHUNK - 1, rows1, gsem1)
        plsc.subcore_barrier()
        pltpu.sync_copy(acc.at[pl.ds(rbase, RPT)], out_at)

    def run_deg_pass(dst_slab, out_at):
        zero_acc()
        pltpu.sync_copy(dst_slab, dsti_all)
        pltpu.sync_copy(ones_hbm, rows0)
        zero_wait()
        plsc.subcore_barrier()

        def dbody(j, carry):
            pltpu.sync_copy(rows0, acc.at[dsti_all.at[j]], add=True)
            return carry

        lax.fori_loop(0, NCHUNK, dbody, 0)
        plsc.subcore_barrier()
        pltpu.sync_copy(acc.at[pl.ds(rbase, RPT)], out_at)

    for g in range(NSEG):
        for p in range(NPCOL):
            run_pass(
                srcidx.at[c, g, p, s],
                dstidx.at[c, g, s],
                s_out.at[c, g, p, pl.ds(rbase, RPT)],
            )
        if with_deg:
            run_deg_pass(
                dstidx.at[c, g, s],
                deg_out.at[c, g, pl.ds(rbase, RPT)],
            )


@functools.cache
def _make_seg_kernel(with_deg):
    out_type = [jax.ShapeDtypeStruct((NCORES, NSEG, NPCOL, NRACC, W),
                                     jnp.float32)]
    if with_deg:
        out_type.append(
            jax.ShapeDtypeStruct((NCORES, NSEG, NRACC, W), jnp.float32))
    scratch = (
        pltpu.VMEM_SHARED((NRACC, W), jnp.float32),
        pltpu.VMEM((NCHUNK, CHUNK), jnp.int32),
        pltpu.VMEM((NCHUNK, CHUNK), jnp.int32),
        pltpu.VMEM((CHUNK, W), jnp.float32),
        pltpu.VMEM((CHUNK, W), jnp.float32),
        pltpu.SemaphoreType.DMA,
        pltpu.SemaphoreType.DMA,
        pltpu.SemaphoreType.DMA,
    )
    mesh = plsc.VectorSubcoreMesh(core_axis_name="c", subcore_axis_name="s")
    body = functools.partial(_seg_body, with_deg)
    return pl.kernel(body, out_type=tuple(out_type), mesh=mesh,
                     scratch_types=scratch,
                     name="segsum_deg" if with_deg else "segsum")


def _combine_w(coeff1, basis1, coeff2, basis2):
    def body(c1_ref, b1_ref, c2_ref, b2_ref, w1_ref, w2_ref):
        for c_ref, b_ref, w_ref in ((c1_ref, b1_ref, w1_ref),
                                    (c2_ref, b2_ref, w2_ref)):
            for r in range(NR):
                acc = c_ref[r, 0] * b_ref[0]
                for b in range(1, NB):
                    acc = acc + c_ref[r, b] * b_ref[b]
                w_ref[r] = acc

    out = jax.ShapeDtypeStruct((NR, FEAT, FEAT), jnp.float32)
    return pl.pallas_call(
        body,
        in_specs=[pl.BlockSpec(memory_space=pltpu.SMEM),
                  pl.BlockSpec((NB, FEAT, FEAT), lambda: (0, 0, 0)),
                  pl.BlockSpec(memory_space=pltpu.SMEM),
                  pl.BlockSpec((NB, FEAT, FEAT), lambda: (0, 0, 0))],
        out_specs=[pl.BlockSpec((NR, FEAT, FEAT), lambda: (0, 0, 0))] * 2,
        out_shape=[out, out],
    )(coeff1, basis1, coeff2, basis2)


def _seg_of(i):
    s = jnp.int32(0)
    for b in SEG_STARTS:
        s = s + (i >= b).astype(jnp.int32)
    return s


def _blk_of(i):
    off = jnp.int32(0)
    starts = (0,) + SEG_STARTS
    for k in range(1, NSEG):
        off = off + (i >= starts[k]).astype(jnp.int32) * (
            starts[k] - starts[k - 1])
    return i - off


def _tc_layer(h_all, s_all, deg_all, wp_all, loop_w, bias, dec):
    def body(h_ref, s_ref, deg_ref, wp_ref, loop_ref, bias_ref, *rest):
        o_ref = rest[-1]
        acc = jnp.dot(h_ref[...], loop_ref[...],
                      preferred_element_type=jnp.float32)
        for a in range(2):
            d = deg_ref[a, 0][:, 0:1]
            inv = 1.0 / jnp.maximum(d, 1.0)
            sa = jnp.concatenate([s_ref[a, 0, p] for p in range(NPCOL)],
                                 axis=1)
            acc = acc + jnp.dot(sa * inv, wp_ref[0, a],
                                preferred_element_type=jnp.float32)
        x = jnp.maximum(acc + bias_ref[...], 0.0)
        if dec is None:
            o_ref[...] = x
        else:
            dw1_ref, db1_ref, dw2_ref, db2_ref = rest[:-1]
            y = jnp.maximum(
                jnp.dot(x, dw1_ref[...], preferred_element_type=jnp.float32)
                + db1_ref[...], 0.0)
            o_ref[...] = (jnp.dot(y, dw2_ref[...],
                                  preferred_element_type=jnp.float32)
                          + db2_ref[...])

    in_specs = [
        pl.BlockSpec((R_BLK, FEAT), lambda i: (i, 0)),
        pl.BlockSpec((NCORES, 1, NPCOL, R_BLK, W),
                     lambda i: (0, _seg_of(i), 0, _blk_of(i), 0)),
        pl.BlockSpec((NCORES, 1, R_BLK, W),
                     lambda i: (0, _seg_of(i), _blk_of(i), 0)),
        pl.BlockSpec((1, NCORES, FEAT, FEAT),
                     lambda i: (_seg_of(i), 0, 0, 0)),
        pl.BlockSpec((FEAT, FEAT), lambda i: (0, 0)),
        pl.BlockSpec((1, FEAT), lambda i: (0, 0)),
    ]
    args = [h_all, s_all, deg_all, wp_all, loop_w, bias]
    out_w = FEAT
    if dec is not None:
        dw1, db1, dw2, db2 = dec
        in_specs += [
            pl.BlockSpec((FEAT, FEAT), lambda i: (0, 0)),
            pl.BlockSpec((1, FEAT), lambda i: (0, 0)),
            pl.BlockSpec((FEAT, OUT), lambda i: (0, 0)),
            pl.BlockSpec((1, OUT), lambda i: (0, 0)),
        ]
        args += [dw1, db1, dw2, db2]
        out_w = OUT

    return pl.pallas_call(
        body,
        grid=(N_TOT // R_BLK,),
        in_specs=in_specs,
        out_specs=pl.BlockSpec((R_BLK, out_w), lambda i: (i, 0)),
        out_shape=jax.ShapeDtypeStruct((N_TOT, out_w), jnp.float32),
    )(*args)


def kernel(h_lnc, h_mi, h_m, src0, dst0, src1, dst1, src2, dst2, src3, dst3,
           src4, dst4, src5, dst5, basis1, coeff1, loop1, bias1, basis2,
           coeff2, loop2, bias2, dec_W1, dec_b1, dec_W2, dec_b2):
    srcs = [src0, src1, src2, src3, src4, src5]
    dsts = [dst0, dst1, dst2, dst3, dst4, dst5]

    W1s, W2s = _combine_w(coeff1, basis1, coeff2, basis2)

    epad = E_PAD - E
    eidx = jnp.arange(E_PAD, dtype=jnp.int32)
    spread = eidx % 2048
    trash = (NRACC - 16) + (eidx & 15)
    src_segs, dst_segs = [], []
    for seg in SEGMENTS:
        src_pair, dst_pair = [], []
        for r, st in zip(seg["rels"], seg["srct"]):
            sp = jnp.concatenate(
                [srcs[r] + OFF[st], jnp.zeros((epad,), jnp.int32)])
            dp = jnp.concatenate(
                [dsts[r], jnp.full((epad,), jnp.int32(1 << 28))])
            lo = seg["dst_lo"]
            ok = (dp >= lo) & (dp < lo + seg["n"])
            dst_pair.append(jnp.where(ok, dp - lo, trash))
            src_pair.append(jnp.where(ok, sp, spread) * NPCOL)
        src_segs.append(jnp.stack(src_pair))
        dst_segs.append(jnp.stack(dst_pair))
    src_base = jnp.stack(src_segs, axis=1)
    srcidx = (jnp.stack([src_base + p for p in range(NPCOL)], axis=2)
              .reshape(NCORES, NSEG, NPCOL, NS, NCHUNK, CHUNK))
    dstidx = jnp.stack(dst_segs, axis=1).reshape(
        NCORES, NSEG, NS, NCHUNK, CHUNK)

    ones_tab = jnp.ones((CHUNK, W), jnp.float32)
    zeros_w = jnp.zeros((RPT, W), jnp.float32)

    wp_all = {}
    for li, Ws in ((0, W1s), (1, W2s)):
        wp_all[li] = jnp.stack(
            [jnp.stack([Ws[seg["rels"][0]], Ws[seg["rels"][1]]])
             for seg in SEGMENTS])

    h_all = jnp.concatenate([h_lnc, h_mi, h_m], axis=0)
    deg_all = None
    for layer in (0, 1):
        table = h_all.reshape(N_TOT * NPCOL, W)
        if layer == 0:
            s_all, deg_all = _make_seg_kernel(True)(
                table, ones_tab, srcidx, dstidx, zeros_w)
        else:
            (s_all,) = _make_seg_kernel(False)(table, srcidx, dstidx, zeros_w)
        loop_w = loop1 if layer == 0 else loop2
        bias = (bias1 if layer == 0 else bias2).reshape(1, FEAT)
        dec = (None if layer == 0 else
               (dec_W1, dec_b1.reshape(1, FEAT), dec_W2,
                dec_b2.reshape(1, OUT)))
        h_all = _tc_layer(h_all, s_all, deg_all, wp_all[layer], loop_w,
                          bias, dec)
    return h_all


# speedup vs baseline: 5.8206x; 1.0140x over previous
import functools

import jax
import jax.numpy as jnp
from jax import lax
from jax.experimental import pallas as pl
from jax.experimental.pallas import tpu as pltpu
from jax.experimental.pallas import tpu_sc as plsc

N_LNC, N_MI, N_M = 10000, 5000, 20000
N_TOT = N_LNC + N_MI + N_M
OFF = {"lnc": 0, "mi": N_LNC, "m": N_LNC + N_MI}
FEAT = 256
OUT = 128
E = 50000
NB = 4
NR = 6

NCORES = 2
NS = 16
W = 128
NPCOL = FEAT // W
CHUNK = 128
EPT = 3200
E_PAD = NS * EPT
NCHUNK = EPT // CHUNK
NRACC = 10112
RPT = NRACC // NS
NSEG = 4
SEGMENTS = (
    dict(rels=(1, 5), srct=("mi", "m"), dst_lo=0, n=N_LNC),
    dict(rels=(0, 3), srct=("lnc", "m"), dst_lo=0, n=N_MI),
    dict(rels=(2, 4), srct=("mi", "lnc"), dst_lo=0, n=10000),
    dict(rels=(2, 4), srct=("mi", "lnc"), dst_lo=10000, n=10000),
)
R_BLK = 1000
SEG_STARTS = (10, 15, 25)


def _seg_body(with_deg, *refs):
    if with_deg:
        (table, ones_hbm, srcidx, dstidx, zeros_hbm,
         s_out, deg_out, acc, srci_all, dsti_all, rows0, rows1,
         gsem0, gsem1, zsem) = refs
    else:
        (table, srcidx, dstidx, zeros_hbm,
         s_out, acc, srci_all, dsti_all, rows0, rows1, gsem0, gsem1,
         zsem) = refs
    c = lax.axis_index("c")
    s = lax.axis_index("s")
    rbase = s * RPT

    def zero_acc():
        pltpu.async_copy(zeros_hbm, acc.at[pl.ds(rbase, RPT)], zsem)

    def zero_wait():
        pltpu.make_async_copy(zeros_hbm, acc.at[pl.ds(rbase, RPT)],
                              zsem).wait()

    def run_pass(src_slab, dst_slab, out_at):
        zero_acc()
        pltpu.sync_copy(src_slab, srci_all)
        pltpu.sync_copy(dst_slab, dsti_all)

        def issue(j, rows, sem):
            pltpu.async_copy(table.at[srci_all.at[j]], rows, sem)

        def wait_scat(j, rows, sem):
            pltpu.make_async_copy(table.at[srci_all.at[j]], rows, sem).wait()
            pltpu.sync_copy(rows, acc.at[dsti_all.at[j]], add=True)

        issue(0, rows0, gsem0)
        issue(1, rows1, gsem1)
        zero_wait()
        plsc.subcore_barrier()
        assert NCHUNK % 2 == 1 and NCHUNK >= 3

        def pbody(t, carry):
            j0 = 2 * t
            wait_scat(j0, rows0, gsem0)
            issue(j0 + 2, rows0, gsem0)
            wait_scat(j0 + 1, rows1, gsem1)
            issue(j0 + 3, rows1, gsem1)
            return carry

        lax.fori_loop(0, (NCHUNK - 3) // 2, pbody, 0)
        wait_scat(NCHUNK - 3, rows0, gsem0)
        issue(NCHUNK - 1, rows0, gsem0)
        wait_scat(NCHUNK - 2, rows1, gsem1)
        wait_scat(NCHUNK - 1, rows0, gsem0)
        plsc.subcore_barrier()
        pltpu.sync_copy(acc.at[pl.ds(rbase, RPT)], out_at)

    def run_deg_pass(dst_slab, out_at):
        zero_acc()
        pltpu.sync_copy(dst_slab, dsti_all)
        pltpu.sync_copy(ones_hbm, rows0)
        zero_wait()
        plsc.subcore_barrier()

        def dbody(j, carry):
            pltpu.async_copy(rows0, acc.at[dsti_all.at[j]], gsem0, add=True)
            return carry

        lax.fori_loop(0, NCHUNK, dbody, 0)

        def dwait(j, carry):
            pltpu.make_async_copy(rows0, acc.at[dsti_all.at[j]],
                                  gsem0).wait()
            return carry

        lax.fori_loop(0, NCHUNK, dwait, 0)
        plsc.subcore_barrier()
        pltpu.sync_copy(acc.at[pl.ds(rbase, RPT)], out_at)

    for g in range(NSEG):
        for p in range(NPCOL):
            run_pass(
                srcidx.at[c, g, p, s],
                dstidx.at[c, g, s],
                s_out.at[c, g, p, pl.ds(rbase, RPT)],
            )
        if with_deg:
            run_deg_pass(
                dstidx.at[c, g, s],
                deg_out.at[c, g, pl.ds(rbase, RPT)],
            )


@functools.cache
def _make_seg_kernel(with_deg):
    out_type = [jax.ShapeDtypeStruct((NCORES, NSEG, NPCOL, NRACC, W),
                                     jnp.float32)]
    if with_deg:
        out_type.append(
            jax.ShapeDtypeStruct((NCORES, NSEG, NRACC, W), jnp.float32))
    scratch = (
        pltpu.VMEM_SHARED((NRACC, W), jnp.float32),
        pltpu.VMEM((NCHUNK, CHUNK), jnp.int32),
        pltpu.VMEM((NCHUNK, CHUNK), jnp.int32),
        pltpu.VMEM((CHUNK, W), jnp.float32),
        pltpu.VMEM((CHUNK, W), jnp.float32),
        pltpu.SemaphoreType.DMA,
        pltpu.SemaphoreType.DMA,
        pltpu.SemaphoreType.DMA,
    )
    mesh = plsc.VectorSubcoreMesh(core_axis_name="c", subcore_axis_name="s")
    body = functools.partial(_seg_body, with_deg)
    return pl.kernel(body, out_type=tuple(out_type), mesh=mesh,
                     scratch_types=scratch,
                     name="segsum_deg" if with_deg else "segsum")


def _combine_w(coeff1, basis1, coeff2, basis2):
    def body(c1_ref, b1_ref, c2_ref, b2_ref, w1_ref, w2_ref):
        for c_ref, b_ref, w_ref in ((c1_ref, b1_ref, w1_ref),
                                    (c2_ref, b2_ref, w2_ref)):
            for r in range(NR):
                acc = c_ref[r, 0] * b_ref[0]
                for b in range(1, NB):
                    acc = acc + c_ref[r, b] * b_ref[b]
                w_ref[r] = acc

    out = jax.ShapeDtypeStruct((NR, FEAT, FEAT), jnp.float32)
    return pl.pallas_call(
        body,
        in_specs=[pl.BlockSpec(memory_space=pltpu.SMEM),
                  pl.BlockSpec((NB, FEAT, FEAT), lambda: (0, 0, 0)),
                  pl.BlockSpec(memory_space=pltpu.SMEM),
                  pl.BlockSpec((NB, FEAT, FEAT), lambda: (0, 0, 0))],
        out_specs=[pl.BlockSpec((NR, FEAT, FEAT), lambda: (0, 0, 0))] * 2,
        out_shape=[out, out],
    )(coeff1, basis1, coeff2, basis2)


def _seg_of(i):
    s = jnp.int32(0)
    for b in SEG_STARTS:
        s = s + (i >= b).astype(jnp.int32)
    return s


def _blk_of(i):
    off = jnp.int32(0)
    starts = (0,) + SEG_STARTS
    for k in range(1, NSEG):
        off = off + (i >= starts[k]).astype(jnp.int32) * (
            starts[k] - starts[k - 1])
    return i - off


def _tc_layer(h_all, s_all, deg_all, wp_all, loop_w, bias, dec):
    def body(h_ref, s_ref, deg_ref, wp_ref, loop_ref, bias_ref, *rest):
        o_ref = rest[-1]
        acc = jnp.dot(h_ref[...], loop_ref[...],
                      preferred_element_type=jnp.float32)
        for a in range(2):
            d = deg_ref[a, 0][:, 0:1]
            inv = 1.0 / jnp.maximum(d, 1.0)
            sa = jnp.concatenate([s_ref[a, 0, p] for p in range(NPCOL)],
                                 axis=1)
            acc = acc + jnp.dot(sa * inv, wp_ref[0, a],
                                preferred_element_type=jnp.float32)
        x = jnp.maximum(acc + bias_ref[...], 0.0)
        if dec is None:
            o_ref[...] = x
        else:
            dw1_ref, db1_ref, dw2_ref, db2_ref = rest[:-1]
            y = jnp.maximum(
                jnp.dot(x, dw1_ref[...], preferred_element_type=jnp.float32)
                + db1_ref[...], 0.0)
            o_ref[...] = (jnp.dot(y, dw2_ref[...],
                                  preferred_element_type=jnp.float32)
                          + db2_ref[...])

    in_specs = [
        pl.BlockSpec((R_BLK, FEAT), lambda i: (i, 0)),
        pl.BlockSpec((NCORES, 1, NPCOL, R_BLK, W),
                     lambda i: (0, _seg_of(i), 0, _blk_of(i), 0)),
        pl.BlockSpec((NCORES, 1, R_BLK, W),
                     lambda i: (0, _seg_of(i), _blk_of(i), 0)),
        pl.BlockSpec((1, NCORES, FEAT, FEAT),
                     lambda i: (_seg_of(i), 0, 0, 0)),
        pl.BlockSpec((FEAT, FEAT), lambda i: (0, 0)),
        pl.BlockSpec((1, FEAT), lambda i: (0, 0)),
    ]
    args = [h_all, s_all, deg_all, wp_all, loop_w, bias]
    out_w = FEAT
    if dec is not None:
        dw1, db1, dw2, db2 = dec
        in_specs += [
            pl.BlockSpec((FEAT, FEAT), lambda i: (0, 0)),
            pl.BlockSpec((1, FEAT), lambda i: (0, 0)),
            pl.BlockSpec((FEAT, OUT), lambda i: (0, 0)),
            pl.BlockSpec((1, OUT), lambda i: (0, 0)),
        ]
        args += [dw1, db1, dw2, db2]
        out_w = OUT

    return pl.pallas_call(
        body,
        grid=(N_TOT // R_BLK,),
        in_specs=in_specs,
        out_specs=pl.BlockSpec((R_BLK, out_w), lambda i: (i, 0)),
        out_shape=jax.ShapeDtypeStruct((N_TOT, out_w), jnp.float32),
    )(*args)


def kernel(h_lnc, h_mi, h_m, src0, dst0, src1, dst1, src2, dst2, src3, dst3,
           src4, dst4, src5, dst5, basis1, coeff1, loop1, bias1, basis2,
           coeff2, loop2, bias2, dec_W1, dec_b1, dec_W2, dec_b2):
    srcs = [src0, src1, src2, src3, src4, src5]
    dsts = [dst0, dst1, dst2, dst3, dst4, dst5]

    W1s, W2s = _combine_w(coeff1, basis1, coeff2, basis2)

    epad = E_PAD - E
    eidx = jnp.arange(E_PAD, dtype=jnp.int32)
    spread = eidx % 2048
    trash = (NRACC - 16) + (eidx & 15)
    src_segs, dst_segs = [], []
    for seg in SEGMENTS:
        src_pair, dst_pair = [], []
        for r, st in zip(seg["rels"], seg["srct"]):
            sp = jnp.concatenate(
                [srcs[r] + OFF[st], jnp.zeros((epad,), jnp.int32)])
            dp = jnp.concatenate(
                [dsts[r], jnp.full((epad,), jnp.int32(1 << 28))])
            lo = seg["dst_lo"]
            ok = (dp >= lo) & (dp < lo + seg["n"])
            dst_pair.append(jnp.where(ok, dp - lo, trash))
            src_pair.append(jnp.where(ok, sp, spread) * NPCOL)
        src_segs.append(jnp.stack(src_pair))
        dst_segs.append(jnp.stack(dst_pair))
    src_base = jnp.stack(src_segs, axis=1)
    srcidx = (jnp.stack([src_base + p for p in range(NPCOL)], axis=2)
              .reshape(NCORES, NSEG, NPCOL, NS, NCHUNK, CHUNK))
    dstidx = jnp.stack(dst_segs, axis=1).reshape(
        NCORES, NSEG, NS, NCHUNK, CHUNK)

    ones_tab = jnp.ones((CHUNK, W), jnp.float32)
    zeros_w = jnp.zeros((RPT, W), jnp.float32)

    wp_all = {}
    for li, Ws in ((0, W1s), (1, W2s)):
        wp_all[li] = jnp.stack(
            [jnp.stack([Ws[seg["rels"][0]], Ws[seg["rels"][1]]])
             for seg in SEGMENTS])

    h_all = jnp.concatenate([h_lnc, h_mi, h_m], axis=0)
    deg_all = None
    for layer in (0, 1):
        table = h_all.reshape(N_TOT * NPCOL, W)
        if layer == 0:
            s_all, deg_all = _make_seg_kernel(True)(
                table, ones_tab, srcidx, dstidx, zeros_w)
        else:
            (s_all,) = _make_seg_kernel(False)(table, srcidx, dstidx, zeros_w)
        loop_w = loop1 if layer == 0 else loop2
        bias = (bias1 if layer == 0 else bias2).reshape(1, FEAT)
        dec = (None if layer == 0 else
               (dec_W1, dec_b1.reshape(1, FEAT), dec_W2,
                dec_b2.reshape(1, OUT)))
        h_all = _tc_layer(h_all, s_all, deg_all, wp_all[layer], loop_w,
                          bias, dec)
    return h_all
